# baseline (speedup 1.0000x reference)
import functools

import jax
import jax.numpy as jnp
from jax import lax
from jax.experimental import pallas as pl
from jax.experimental.pallas import tpu as pltpu
from jax.experimental.pallas import tpu_sc as plsc

N = 100000
E = 1600000
NC = 2
NS = 16
NW = NC * NS
NP = 100352
EP = 1638400
EW = EP // NW
C = 1024
NCH = EW // C
CE = 400
EWE = E // NW
NCHE = EWE // CE
NSL = NP // NS
NB = 128
NBL = NSL // NB

_mesh = plsc.VectorSubcoreMesh(
    core_axis_name="c", subcore_axis_name="s", num_cores=NC, num_subcores=NS)
_params = pltpu.CompilerParams(
    use_tc_tiling_on_sc=False, needs_layout_passes=False)


def _rsqrt16(x):
    i = plsc.bitcast(x, jnp.int32)
    i = 0x5F3759DF - lax.shift_right_logical(i, 1)
    y = plsc.bitcast(i, jnp.float32)
    for _ in range(3):
        y = y * (1.5 - 0.5 * x * y * y)
    return y


@functools.partial(
    pl.kernel,
    out_type=jax.ShapeDtypeStruct((NC, NP), jnp.float32),
    mesh=_mesh,
    compiler_params=_params,
    scratch_types=[
        pltpu.VMEM((C // 128, 128), jnp.int32),
        pltpu.VMEM((128,), jnp.float32),
        pltpu.VMEM_SHARED((NP,), jnp.float32),
        pltpu.SemaphoreType.DMA,
    ],
)
def _deg_kernel(dst2d, zeros1, parts, idx2d, ones_v, acc, sem):
    c = lax.axis_index("c")
    s = lax.axis_index("s")
    w = c * NS + s

    def fill_ones(i, _):
        ones_v[pl.ds(pl.multiple_of(i * 16, 16), 16)] = jnp.ones((16,), jnp.float32)
        return 0
    lax.fori_loop(0, 8, fill_ones, 0)
    sl = pl.ds(pl.multiple_of(s * NSL, NSL), NSL)
    pltpu.sync_copy(zeros1.at[sl], acc.at[sl])
    plsc.subcore_barrier()

    def chunk(i, _):
        row_base = pl.multiple_of(w * (EW // 128) + i * (C // 128), C // 128)
        pltpu.sync_copy(dst2d.at[pl.ds(row_base, C // 128)], idx2d)

        cps = [pltpu.async_copy(ones_v, acc.at[idx2d.at[r]], sem, add=True)
               for r in range(C // 128)]
        for cp in cps:
            cp.wait()
        return 0
    lax.fori_loop(0, NCH, chunk, 0)
    plsc.subcore_barrier()
    pltpu.sync_copy(acc.at[sl], parts.at[c, sl])


@functools.partial(
    pl.kernel,
    out_type=(
        jax.ShapeDtypeStruct((NP,), jnp.float32),
        jax.ShapeDtypeStruct((NP,), jnp.float32),
        jax.ShapeDtypeStruct((NC, NP), jnp.float32),
    ),
    mesh=_mesh,
    compiler_params=_params,
    scratch_types=[
        pltpu.VMEM((NP,), jnp.float32),
        pltpu.VMEM((C,), jnp.int32),
        pltpu.VMEM((C // 128, 128), jnp.int32),
        pltpu.VMEM((C,), jnp.float32),
        pltpu.VMEM((NB,), jnp.float32),
        pltpu.VMEM((NB,), jnp.float32),
        pltpu.VMEM((NB,), jnp.float32),
        pltpu.VMEM((NB,), jnp.float32),
        pltpu.VMEM((NB,), jnp.float32),
        pltpu.VMEM_SHARED((NP,), jnp.float32),
        pltpu.SemaphoreType.DMA,
    ],
)
def _seg1_kernel(src1d, dst2d, x1d, parts0, zeros1, dinv, y0, parts1,
                 table, sidx, idx2d, upd, xb, p0b, p1b, db, yb, acc, sem):
    c = lax.axis_index("c")
    s = lax.axis_index("s")
    w = c * NS + s
    sl = pl.ds(pl.multiple_of(s * NSL, NSL), NSL)
    pltpu.sync_copy(zeros1.at[sl], acc.at[sl])

    def blk(b, _):
        off = pl.multiple_of(s * NSL + b * NB, NB)
        pltpu.sync_copy(x1d.at[pl.ds(off, NB)], xb)
        pltpu.sync_copy(parts0.at[0, pl.ds(off, NB)], p0b)
        pltpu.sync_copy(parts0.at[1, pl.ds(off, NB)], p1b)

        def grp(g, _):
            gs = pl.ds(pl.multiple_of(g * 16, 16), 16)
            deg = p0b[gs] + p1b[gs] + 1.0
            dv = _rsqrt16(deg)
            db[gs] = dv
            yb[gs] = dv * xb[gs]
            return 0
        lax.fori_loop(0, NB // 16, grp, 0)
        pltpu.sync_copy(db, dinv.at[pl.ds(off, NB)])
        pltpu.sync_copy(yb, y0.at[pl.ds(off, NB)])
        return 0
    lax.fori_loop(0, NBL, blk, 0)
    plsc.subcore_barrier()

    pltpu.sync_copy(y0, table)

    def chunk(i, _):
        base = pl.multiple_of(w * EW + i * C, C)
        pltpu.sync_copy(src1d.at[pl.ds(base, C)], sidx)
        pltpu.sync_copy(dst2d.at[pl.ds(pl.multiple_of(base // 128, C // 128), C // 128)], idx2d)

        def grp(m, _):
            gs = pl.ds(pl.multiple_of(m * 16, 16), 16)
            upd[gs] = plsc.load_gather(table, [sidx[gs]])
            return 0
        lax.fori_loop(0, C // 16, grp, 0)

        cps = [pltpu.async_copy(upd.at[pl.ds(r * 128, 128)],
                                acc.at[idx2d.at[r]], sem, add=True)
               for r in range(C // 128)]
        for cp in cps:
            cp.wait()
        return 0
    lax.fori_loop(0, NCH, chunk, 0)
    plsc.subcore_barrier()
    pltpu.sync_copy(acc.at[sl], parts1.at[c, sl])


def _make_seg16(nk):
    scalar_head = nk == 1
    if scalar_head:
        head_scr = [pltpu.VMEM((NB,), jnp.float32)] * 3
        yprev_t = jax.ShapeDtypeStruct((NP,), jnp.float32)
        pin_t = jax.ShapeDtypeStruct((NC, NP), jnp.float32)
    else:
        head_scr = [pltpu.VMEM((NB, 16), jnp.float32)] * 3
        yprev_t = jax.ShapeDtypeStruct((NP, 16), jnp.float32)
        pin_t = jax.ShapeDtypeStruct((NC, NP, 16), jnp.float32)
    del yprev_t, pin_t

    @functools.partial(
        pl.kernel,
        out_type=(
            jax.ShapeDtypeStruct((NP, 16), jnp.float32),
            jax.ShapeDtypeStruct((NC, NP, 16), jnp.float32),
        ),
        mesh=_mesh,
        compiler_params=_params,
        scratch_types=[
            pltpu.VMEM((C,), jnp.int32),
            pltpu.VMEM((C // 128, 128), jnp.int32),
            pltpu.VMEM((C, 16), jnp.float32),
            pltpu.VMEM((16, 16), jnp.float32),
            pltpu.VMEM((16,), jnp.float32),
            *head_scr,
            pltpu.VMEM((NB,), jnp.float32),
            pltpu.VMEM((NB, 16), jnp.float32),
            pltpu.VMEM_SHARED((NP, 16), jnp.float32),
            pltpu.SemaphoreType.DMA,
        ],
    )
    def seg(src1d, dst2d, wmat, bvec, pin, yprev, dinv, zeros16, ytab, parts,
            sidx, idx2d, rows, wv, bv, p0b, p1b, ypb, db, yb, acc, sem):
        c = lax.axis_index("c")
        s = lax.axis_index("s")
        w = c * NS + s
        sl = pl.ds(pl.multiple_of(s * NSL, NSL), NSL)
        pltpu.sync_copy(zeros16.at[sl], acc.at[sl])
        pltpu.sync_copy(wmat, wv)
        pltpu.sync_copy(bvec, bv)

        def blk(b, _):
            off = pl.multiple_of(s * NSL + b * NB, NB)
            cps = [
                pltpu.async_copy(pin.at[0, pl.ds(off, NB)], p0b, sem),
                pltpu.async_copy(pin.at[1, pl.ds(off, NB)], p1b, sem),
                pltpu.async_copy(yprev.at[pl.ds(off, NB)], ypb, sem),
                pltpu.async_copy(dinv.at[pl.ds(off, NB)], db, sem),
            ]
            for cp in cps:
                cp.wait()
            bvv = bv[pl.ds(0, 16)]

            if scalar_head:
                w1row = wv[0, 0:16]

                def grp(g, _):
                    gs = pl.ds(pl.multiple_of(g * 16, 16), 16)
                    dvv = db[gs]
                    a0v = dvv * (p0b[gs] + p1b[gs] + ypb[gs])
                    for j in range(16):
                        h = jnp.maximum(a0v[j] * w1row + bvv, 0.0)
                        yb[g * 16 + j, 0:16] = dvv[j] * h
                    return 0
                lax.fori_loop(0, NB // 16, grp, 0)
            else:
                wrows = [wv[k, 0:16] for k in range(8)]

                def grp(g, _):
                    dvv = db[pl.ds(pl.multiple_of(g * 16, 16), 16)]
                    for j in range(16):
                        n = g * 16 + j
                        ar = dvv[j] * (p0b[n, 0:16] + p1b[n, 0:16]
                                       + ypb[n, 0:16])
                        h = bvv
                        for k in range(8):
                            h = h + ar[k] * wrows[k]
                        yb[n, 0:16] = dvv[j] * jnp.maximum(h, 0.0)
                    return 0
                lax.fori_loop(0, NB // 16, grp, 0)
            pltpu.sync_copy(yb, ytab.at[pl.ds(off, NB)])
            return 0
        lax.fori_loop(0, NBL, blk, 0)
        plsc.subcore_barrier()

        def chunk(i, _):
            base = pl.multiple_of(w * EW + i * C, C)
            pltpu.sync_copy(src1d.at[pl.ds(base, C)], sidx)
            pltpu.sync_copy(dst2d.at[pl.ds(pl.multiple_of(base // 128, C // 128), C // 128)], idx2d)
            pltpu.async_copy(ytab.at[sidx], rows, sem).wait()

            cps = [pltpu.async_copy(rows.at[pl.ds(r * 128, 128)],
                                    acc.at[idx2d.at[r]], sem, add=True)
                   for r in range(C // 128)]
            for cp in cps:
                cp.wait()
            return 0
        lax.fori_loop(0, NCH, chunk, 0)
        plsc.subcore_barrier()
        pltpu.sync_copy(acc.at[sl], parts.at[c, sl])

    return seg


_seg16_l2 = _make_seg16(1)
_seg16_l3 = _make_seg16(2)


@functools.partial(
    pl.kernel,
    out_type=(
        jax.ShapeDtypeStruct((NP, 16), jnp.float32),
        jax.ShapeDtypeStruct((NP, 16), jnp.float32),
        jax.ShapeDtypeStruct((E,), jnp.float32),
    ),
    mesh=_mesh,
    compiler_params=_params,
    scratch_types=[
        pltpu.VMEM((2, CE), jnp.int32),
        pltpu.VMEM((2, CE), jnp.int32),
        pltpu.VMEM((2, CE, 16), jnp.float32),
        pltpu.VMEM((2, CE, 16), jnp.float32),
        pltpu.VMEM((CE,), jnp.float32),
        pltpu.VMEM((16, 16), jnp.float32),
        pltpu.VMEM((32,), jnp.float32),
        pltpu.VMEM((NB, 16), jnp.float32),
        pltpu.VMEM((NB, 16), jnp.float32),
        pltpu.VMEM((NB, 16), jnp.float32),
        pltpu.VMEM((NB,), jnp.float32),
        pltpu.VMEM((NB, 16), jnp.float32),
        pltpu.VMEM((NB, 16), jnp.float32),
        pltpu.SemaphoreType.DMA,
        pltpu.SemaphoreType.DMA,
    ],
)
def _edge_kernel(src1d, dst1d, wmat, bvec, pin, yprev, dinv, gtab, vtab, out,
                 sidx, didx, gr, ar, outv, wv, bv, p0b, p1b, ypb, db, gb, vb,
                 sem1, sem2):
    c = lax.axis_index("c")
    s = lax.axis_index("s")
    w = c * NS + s
    lanes = lax.iota(jnp.int32, 16)
    pltpu.sync_copy(wmat, wv)
    pltpu.sync_copy(bvec, bv)

    def blk(b, _):
        off = pl.multiple_of(s * NSL + b * NB, NB)
        cps = [
            pltpu.async_copy(pin.at[0, pl.ds(off, NB)], p0b, sem1),
            pltpu.async_copy(pin.at[1, pl.ds(off, NB)], p1b, sem1),
            pltpu.async_copy(yprev.at[pl.ds(off, NB)], ypb, sem1),
            pltpu.async_copy(dinv.at[pl.ds(off, NB)], db, sem1),
        ]
        for cp in cps:
            cp.wait()
        uv = bv[pl.ds(0, 16)]
        mrows = [wv[k, 0:16] for k in range(16)]

        def grp(g, _):
            dvv = db[pl.ds(pl.multiple_of(g * 16, 16), 16)]
            for j in range(16):
                n = g * 16 + j
                vr = dvv[j] * (p0b[n, 0:16] + p1b[n, 0:16] + ypb[n, 0:16]) + uv
                acc = vr[0] * mrows[0]
                for k in range(1, 16):
                    acc = acc + vr[k] * mrows[k]
                vb[n, 0:16] = vr
                gb[n, 0:16] = acc
            return 0
        lax.fori_loop(0, NB // 16, grp, 0)
        pltpu.sync_copy(vb, vtab.at[pl.ds(off, NB)])
        pltpu.sync_copy(gb, gtab.at[pl.ds(off, NB)])
        return 0
    lax.fori_loop(0, NBL, blk, 0)
    plsc.subcore_barrier()
    c1 = bv[pl.ds(16, 16)][0]

    def fire(i, p):
        base = pl.multiple_of(w * EWE + i * CE, 8)
        pltpu.sync_copy(src1d.at[pl.ds(base, CE)], sidx.at[p])
        pltpu.sync_copy(dst1d.at[pl.ds(base, CE)], didx.at[p])
        pltpu.async_copy(gtab.at[sidx.at[p]], gr.at[p], sem1)
        pltpu.async_copy(vtab.at[didx.at[p]], ar.at[p], sem2)

    def consume(i, p):
        pltpu.make_async_copy(gtab.at[sidx.at[p]], gr.at[p], sem1).wait()
        pltpu.make_async_copy(vtab.at[didx.at[p]], ar.at[p], sem2).wait()
        base = pl.multiple_of(w * EWE + i * CE, 8)

        def grp(m, _):
            res = jnp.zeros((16,), jnp.float32)
            for j in range(16):
                e = m * 16 + j
                q = gr[p, e, 0:16] * ar[p, e, 0:16]
                res = jnp.where(lanes == j, jnp.sum(q), res)
            outv[pl.ds(pl.multiple_of(m * 16, 16), 16)] = (
                1.0 / (1.0 + jnp.exp(-(res + c1))))
            return 0
        lax.fori_loop(0, CE // 16, grp, 0)
        pltpu.sync_copy(outv, out.at[pl.ds(base, CE)])

    fire(0, 0)

    def chunk(i, _):
        even = i % 2 == 0

        @pl.when(jnp.logical_and(even, i + 1 < NCHE))
        def _():
            fire(i + 1, 1)

        @pl.when(jnp.logical_and(jnp.logical_not(even), i + 1 < NCHE))
        def _():
            fire(i + 1, 0)

        @pl.when(even)
        def _():
            consume(i, 0)

        @pl.when(jnp.logical_not(even))
        def _():
            consume(i, 1)
        return 0
    lax.fori_loop(0, NCHE, chunk, 0)


def kernel(x, edge_index, W1, b1, W2, b2, W3, b3):
    x1d = jnp.pad(x[:, 0], (0, NP - N))
    src = edge_index[0]
    dst = edge_index[1]
    fill = (jnp.arange(EP - E, dtype=jnp.int32) % (NP - N)) + N
    src_p = jnp.concatenate([src, fill])
    dst_p = jnp.concatenate([dst, fill])
    dst2d = dst_p.reshape(EP // 128, 128)
    zeros1 = jnp.zeros((NP,), jnp.float32)
    zeros16 = jnp.zeros((NP, 16), jnp.float32)
    w1p = jnp.zeros((16, 16), jnp.float32).at[0, :8].set(W1[0])
    b1p = jnp.zeros((16,), jnp.float32).at[:8].set(b1)
    w2p = jnp.zeros((16, 16), jnp.float32).at[:8, :].set(W2)

    parts0 = _deg_kernel(dst2d, zeros1)
    dinv, y0, parts1 = _seg1_kernel(src_p, dst2d, x1d, parts0, zeros1)
    y1p, parts2 = _seg16_l2(src_p, dst2d, w1p, b1p, parts1, y0, dinv, zeros16)
    y2, parts3 = _seg16_l3(src_p, dst2d, w2p, b2, parts2, y1p, dinv, zeros16)
    m3 = W3 @ W3.T
    wb3 = W3 @ b3
    u3 = jnp.linalg.solve(m3, wb3)
    c1 = b3 @ b3 - u3 @ (m3 @ u3)
    uc = jnp.concatenate([u3, jnp.full((16,), c1, jnp.float32)])
    _, _, scores = _edge_kernel(src, dst, m3, uc, parts3, y2, dinv)
    return scores


# speedup vs baseline: 1.0109x; 1.0109x over previous
import functools

import jax
import jax.numpy as jnp
from jax import lax
from jax.experimental import pallas as pl
from jax.experimental.pallas import tpu as pltpu
from jax.experimental.pallas import tpu_sc as plsc

N = 100000
E = 1600000
NC = 2
NS = 16
NW = NC * NS
NP = 100352
EP = 1638400
EW = EP // NW
C = 1024
NCH = EW // C
CE = 2000
EWE = E // NW
NCHE = EWE // CE
NSL = NP // NS
NB = 128
NBL = NSL // NB

_mesh = plsc.VectorSubcoreMesh(
    core_axis_name="c", subcore_axis_name="s", num_cores=NC, num_subcores=NS)
_params = pltpu.CompilerParams(
    use_tc_tiling_on_sc=False, needs_layout_passes=False)


def _rsqrt16(x):
    i = plsc.bitcast(x, jnp.int32)
    i = 0x5F3759DF - lax.shift_right_logical(i, 1)
    y = plsc.bitcast(i, jnp.float32)
    for _ in range(3):
        y = y * (1.5 - 0.5 * x * y * y)
    return y


@functools.partial(
    pl.kernel,
    out_type=jax.ShapeDtypeStruct((NC, NP), jnp.float32),
    mesh=_mesh,
    compiler_params=_params,
    scratch_types=[
        pltpu.VMEM((C // 128, 128), jnp.int32),
        pltpu.VMEM((128,), jnp.float32),
        pltpu.VMEM_SHARED((NP,), jnp.float32),
        pltpu.SemaphoreType.DMA,
    ],
)
def _deg_kernel(dst2d, zeros1, parts, idx2d, ones_v, acc, sem):
    c = lax.axis_index("c")
    s = lax.axis_index("s")
    w = c * NS + s

    def fill_ones(i, _):
        ones_v[pl.ds(pl.multiple_of(i * 16, 16), 16)] = jnp.ones((16,), jnp.float32)
        return 0
    lax.fori_loop(0, 8, fill_ones, 0)
    sl = pl.ds(pl.multiple_of(s * NSL, NSL), NSL)
    pltpu.sync_copy(zeros1.at[sl], acc.at[sl])
    plsc.subcore_barrier()

    def chunk(i, _):
        row_base = pl.multiple_of(w * (EW // 128) + i * (C // 128), C // 128)
        pltpu.sync_copy(dst2d.at[pl.ds(row_base, C // 128)], idx2d)

        cps = [pltpu.async_copy(ones_v, acc.at[idx2d.at[r]], sem, add=True)
               for r in range(C // 128)]
        for cp in cps:
            cp.wait()
        return 0
    lax.fori_loop(0, NCH, chunk, 0)
    plsc.subcore_barrier()
    pltpu.sync_copy(acc.at[sl], parts.at[c, sl])


@functools.partial(
    pl.kernel,
    out_type=(
        jax.ShapeDtypeStruct((NP,), jnp.float32),
        jax.ShapeDtypeStruct((NP,), jnp.float32),
        jax.ShapeDtypeStruct((NC, NP), jnp.float32),
    ),
    mesh=_mesh,
    compiler_params=_params,
    scratch_types=[
        pltpu.VMEM((NP,), jnp.float32),
        pltpu.VMEM((C,), jnp.int32),
        pltpu.VMEM((C // 128, 128), jnp.int32),
        pltpu.VMEM((C,), jnp.float32),
        pltpu.VMEM((NB,), jnp.float32),
        pltpu.VMEM((NB,), jnp.float32),
        pltpu.VMEM((NB,), jnp.float32),
        pltpu.VMEM((NB,), jnp.float32),
        pltpu.VMEM((NB,), jnp.float32),
        pltpu.VMEM_SHARED((NP,), jnp.float32),
        pltpu.SemaphoreType.DMA,
    ],
)
def _seg1_kernel(src1d, dst2d, x1d, parts0, zeros1, dinv, y0, parts1,
                 table, sidx, idx2d, upd, xb, p0b, p1b, db, yb, acc, sem):
    c = lax.axis_index("c")
    s = lax.axis_index("s")
    w = c * NS + s
    sl = pl.ds(pl.multiple_of(s * NSL, NSL), NSL)
    pltpu.sync_copy(zeros1.at[sl], acc.at[sl])

    def blk(b, _):
        off = pl.multiple_of(s * NSL + b * NB, NB)
        pltpu.sync_copy(x1d.at[pl.ds(off, NB)], xb)
        pltpu.sync_copy(parts0.at[0, pl.ds(off, NB)], p0b)
        pltpu.sync_copy(parts0.at[1, pl.ds(off, NB)], p1b)

        def grp(g, _):
            gs = pl.ds(pl.multiple_of(g * 16, 16), 16)
            deg = p0b[gs] + p1b[gs] + 1.0
            dv = _rsqrt16(deg)
            db[gs] = dv
            yb[gs] = dv * xb[gs]
            return 0
        lax.fori_loop(0, NB // 16, grp, 0)
        pltpu.sync_copy(db, dinv.at[pl.ds(off, NB)])
        pltpu.sync_copy(yb, y0.at[pl.ds(off, NB)])
        return 0
    lax.fori_loop(0, NBL, blk, 0)
    plsc.subcore_barrier()

    pltpu.sync_copy(y0, table)

    def chunk(i, _):
        base = pl.multiple_of(w * EW + i * C, C)
        pltpu.sync_copy(src1d.at[pl.ds(base, C)], sidx)
        pltpu.sync_copy(dst2d.at[pl.ds(pl.multiple_of(base // 128, C // 128), C // 128)], idx2d)

        def grp(m, _):
            gs = pl.ds(pl.multiple_of(m * 16, 16), 16)
            upd[gs] = plsc.load_gather(table, [sidx[gs]])
            return 0
        lax.fori_loop(0, C // 16, grp, 0)

        cps = [pltpu.async_copy(upd.at[pl.ds(r * 128, 128)],
                                acc.at[idx2d.at[r]], sem, add=True)
               for r in range(C // 128)]
        for cp in cps:
            cp.wait()
        return 0
    lax.fori_loop(0, NCH, chunk, 0)
    plsc.subcore_barrier()
    pltpu.sync_copy(acc.at[sl], parts1.at[c, sl])


def _make_seg16(nk):
    scalar_head = nk == 1
    if scalar_head:
        head_scr = [pltpu.VMEM((NB,), jnp.float32)] * 3
        yprev_t = jax.ShapeDtypeStruct((NP,), jnp.float32)
        pin_t = jax.ShapeDtypeStruct((NC, NP), jnp.float32)
    else:
        head_scr = [pltpu.VMEM((NB, 16), jnp.float32)] * 3
        yprev_t = jax.ShapeDtypeStruct((NP, 16), jnp.float32)
        pin_t = jax.ShapeDtypeStruct((NC, NP, 16), jnp.float32)
    del yprev_t, pin_t

    @functools.partial(
        pl.kernel,
        out_type=(
            jax.ShapeDtypeStruct((NP, 16), jnp.float32),
            jax.ShapeDtypeStruct((NC, NP, 16), jnp.float32),
        ),
        mesh=_mesh,
        compiler_params=_params,
        scratch_types=[
            pltpu.VMEM((C,), jnp.int32),
            pltpu.VMEM((C // 128, 128), jnp.int32),
            pltpu.VMEM((C, 16), jnp.float32),
            pltpu.VMEM((16, 16), jnp.float32),
            pltpu.VMEM((16,), jnp.float32),
            *head_scr,
            pltpu.VMEM((NB,), jnp.float32),
            pltpu.VMEM((NB, 16), jnp.float32),
            pltpu.VMEM_SHARED((NP, 16), jnp.float32),
            pltpu.SemaphoreType.DMA,
        ],
    )
    def seg(src1d, dst2d, wmat, bvec, pin, yprev, dinv, zeros16, ytab, parts,
            sidx, idx2d, rows, wv, bv, p0b, p1b, ypb, db, yb, acc, sem):
        c = lax.axis_index("c")
        s = lax.axis_index("s")
        w = c * NS + s
        sl = pl.ds(pl.multiple_of(s * NSL, NSL), NSL)
        pltpu.sync_copy(zeros16.at[sl], acc.at[sl])
        pltpu.sync_copy(wmat, wv)
        pltpu.sync_copy(bvec, bv)

        def blk(b, _):
            off = pl.multiple_of(s * NSL + b * NB, NB)
            cps = [
                pltpu.async_copy(pin.at[0, pl.ds(off, NB)], p0b, sem),
                pltpu.async_copy(pin.at[1, pl.ds(off, NB)], p1b, sem),
                pltpu.async_copy(yprev.at[pl.ds(off, NB)], ypb, sem),
                pltpu.async_copy(dinv.at[pl.ds(off, NB)], db, sem),
            ]
            for cp in cps:
                cp.wait()
            bvv = bv[pl.ds(0, 16)]

            if scalar_head:
                w1row = wv[0, 0:16]

                def grp(g, _):
                    gs = pl.ds(pl.multiple_of(g * 16, 16), 16)
                    dvv = db[gs]
                    a0v = dvv * (p0b[gs] + p1b[gs] + ypb[gs])
                    for j in range(16):
                        h = jnp.maximum(a0v[j] * w1row + bvv, 0.0)
                        yb[g * 16 + j, 0:16] = dvv[j] * h
                    return 0
                lax.fori_loop(0, NB // 16, grp, 0)
            else:
                wrows = [wv[k, 0:16] for k in range(8)]

                def grp(g, _):
                    dvv = db[pl.ds(pl.multiple_of(g * 16, 16), 16)]
                    for j in range(16):
                        n = g * 16 + j
                        ar = dvv[j] * (p0b[n, 0:16] + p1b[n, 0:16]
                                       + ypb[n, 0:16])
                        h = bvv
                        for k in range(8):
                            h = h + ar[k] * wrows[k]
                        yb[n, 0:16] = dvv[j] * jnp.maximum(h, 0.0)
                    return 0
                lax.fori_loop(0, NB // 16, grp, 0)
            pltpu.sync_copy(yb, ytab.at[pl.ds(off, NB)])
            return 0
        lax.fori_loop(0, NBL, blk, 0)
        plsc.subcore_barrier()

        def chunk(i, _):
            base = pl.multiple_of(w * EW + i * C, C)
            pltpu.sync_copy(src1d.at[pl.ds(base, C)], sidx)
            pltpu.sync_copy(dst2d.at[pl.ds(pl.multiple_of(base // 128, C // 128), C // 128)], idx2d)
            pltpu.async_copy(ytab.at[sidx], rows, sem).wait()

            cps = [pltpu.async_copy(rows.at[pl.ds(r * 128, 128)],
                                    acc.at[idx2d.at[r]], sem, add=True)
                   for r in range(C // 128)]
            for cp in cps:
                cp.wait()
            return 0
        lax.fori_loop(0, NCH, chunk, 0)
        plsc.subcore_barrier()
        pltpu.sync_copy(acc.at[sl], parts.at[c, sl])

    return seg


_seg16_l2 = _make_seg16(1)
_seg16_l3 = _make_seg16(2)


@functools.partial(
    pl.kernel,
    out_type=(
        jax.ShapeDtypeStruct((NP, 16), jnp.float32),
        jax.ShapeDtypeStruct((NP, 16), jnp.float32),
        jax.ShapeDtypeStruct((E,), jnp.float32),
    ),
    mesh=_mesh,
    compiler_params=_params,
    scratch_types=[
        pltpu.VMEM((CE,), jnp.int32),
        pltpu.VMEM((CE,), jnp.int32),
        pltpu.VMEM((CE, 16), jnp.float32),
        pltpu.VMEM((CE, 16), jnp.float32),
        pltpu.VMEM((CE,), jnp.float32),
        pltpu.VMEM((16, 16), jnp.float32),
        pltpu.VMEM((32,), jnp.float32),
        pltpu.VMEM((NB, 16), jnp.float32),
        pltpu.VMEM((NB, 16), jnp.float32),
        pltpu.VMEM((NB, 16), jnp.float32),
        pltpu.VMEM((NB,), jnp.float32),
        pltpu.VMEM((NB, 16), jnp.float32),
        pltpu.VMEM((NB, 16), jnp.float32),
        pltpu.SemaphoreType.DMA,
        pltpu.SemaphoreType.DMA,
    ],
)
def _edge_kernel(src1d, dst1d, wmat, bvec, pin, yprev, dinv, gtab, vtab, out,
                 sidx, didx, gr, ar, outv, wv, bv, p0b, p1b, ypb, db, gb, vb,
                 sem1, sem2):
    c = lax.axis_index("c")
    s = lax.axis_index("s")
    w = c * NS + s
    lanes = lax.iota(jnp.int32, 16)
    pltpu.sync_copy(wmat, wv)
    pltpu.sync_copy(bvec, bv)

    def blk(b, _):
        off = pl.multiple_of(s * NSL + b * NB, NB)
        cps = [
            pltpu.async_copy(pin.at[0, pl.ds(off, NB)], p0b, sem1),
            pltpu.async_copy(pin.at[1, pl.ds(off, NB)], p1b, sem1),
            pltpu.async_copy(yprev.at[pl.ds(off, NB)], ypb, sem1),
            pltpu.async_copy(dinv.at[pl.ds(off, NB)], db, sem1),
        ]
        for cp in cps:
            cp.wait()
        uv = bv[pl.ds(0, 16)]
        mrows = [wv[k, 0:16] for k in range(16)]

        def grp(g, _):
            dvv = db[pl.ds(pl.multiple_of(g * 16, 16), 16)]
            for j in range(16):
                n = g * 16 + j
                vr = dvv[j] * (p0b[n, 0:16] + p1b[n, 0:16] + ypb[n, 0:16]) + uv
                acc = vr[0] * mrows[0]
                for k in range(1, 16):
                    acc = acc + vr[k] * mrows[k]
                vb[n, 0:16] = vr
                gb[n, 0:16] = acc
            return 0
        lax.fori_loop(0, NB // 16, grp, 0)
        pltpu.sync_copy(vb, vtab.at[pl.ds(off, NB)])
        pltpu.sync_copy(gb, gtab.at[pl.ds(off, NB)])
        return 0
    lax.fori_loop(0, NBL, blk, 0)
    plsc.subcore_barrier()
    c1 = bv[pl.ds(16, 16)][0]

    def chunk(i, _):
        base = pl.multiple_of(w * EWE + i * CE, 8)
        pltpu.sync_copy(src1d.at[pl.ds(base, CE)], sidx)
        pltpu.sync_copy(dst1d.at[pl.ds(base, CE)], didx)
        cp1 = pltpu.async_copy(gtab.at[sidx], gr, sem1)
        cp2 = pltpu.async_copy(vtab.at[didx], ar, sem2)
        cp1.wait()
        cp2.wait()

        def grp(m, _):
            res = jnp.zeros((16,), jnp.float32)
            for j in range(16):
                e = m * 16 + j
                q = gr[e, 0:16] * ar[e, 0:16]
                res = jnp.where(lanes == j, jnp.sum(q), res)
            outv[pl.ds(pl.multiple_of(m * 16, 16), 16)] = (
                1.0 / (1.0 + jnp.exp(-(res + c1))))
            return 0
        lax.fori_loop(0, CE // 16, grp, 0)
        pltpu.sync_copy(outv, out.at[pl.ds(base, CE)])
        return 0
    lax.fori_loop(0, NCHE, chunk, 0)


def kernel(x, edge_index, W1, b1, W2, b2, W3, b3):
    x1d = jnp.pad(x[:, 0], (0, NP - N))
    src = edge_index[0]
    dst = edge_index[1]
    fill = (jnp.arange(EP - E, dtype=jnp.int32) % (NP - N)) + N
    src_p = jnp.concatenate([src, fill])
    dst_p = jnp.concatenate([dst, fill])
    dst2d = dst_p.reshape(EP // 128, 128)
    zeros1 = jnp.zeros((NP,), jnp.float32)
    zeros16 = jnp.zeros((NP, 16), jnp.float32)
    w1p = jnp.zeros((16, 16), jnp.float32).at[0, :8].set(W1[0])
    b1p = jnp.zeros((16,), jnp.float32).at[:8].set(b1)
    w2p = jnp.zeros((16, 16), jnp.float32).at[:8, :].set(W2)

    parts0 = _deg_kernel(dst2d, zeros1)
    dinv, y0, parts1 = _seg1_kernel(src_p, dst2d, x1d, parts0, zeros1)
    y1p, parts2 = _seg16_l2(src_p, dst2d, w1p, b1p, parts1, y0, dinv, zeros16)
    y2, parts3 = _seg16_l3(src_p, dst2d, w2p, b2, parts2, y1p, dinv, zeros16)
    m3 = W3 @ W3.T
    wb3 = W3 @ b3
    u3 = jnp.linalg.solve(m3, wb3)
    c1 = b3 @ b3 - u3 @ (m3 @ u3)
    uc = jnp.concatenate([u3, jnp.full((16,), c1, jnp.float32)])
    _, _, scores = _edge_kernel(src, dst, m3, uc, parts3, y2, dinv)
    return scores
